# Initial kernel scaffold; baseline (speedup 1.0000x reference)
import jax
import jax.numpy as jnp
from jax.experimental import pallas as pl


def kernel(x, edge_index, W_agg, b_agg, W_agg_r, b_agg_r, Wih, Whh, bih, bhh, Wih_r, Whh_r, bih_r, bhh_r, ln_g, ln_b):
    raise NotImplementedError("write your pallas kernel here")


# speedup vs baseline: 3.1220x; 3.1220x over previous
import functools

import jax
import jax.numpy as jnp
from jax import lax
from jax.experimental import pallas as pl
from jax.experimental.pallas import tpu as pltpu
from jax.experimental.pallas import tpu_sc as plsc

N = 10000
E = 320000
H = 128
F = 128
ROUNDS = 3

NC = 2
NS = 16
NW = NC * NS
EPW = E // NW
CHUNK = 80
NCHUNK = EPW // CHUNK
STRIPE = 624
TAIL = N - NS * STRIPE


def _sc_scatter_make():
    mesh = plsc.VectorSubcoreMesh(core_axis_name="c", subcore_axis_name="s")

    @functools.partial(
        pl.kernel,
        out_type=jax.ShapeDtypeStruct((NC * N, H), jnp.float32),
        mesh=mesh,
        scratch_types=[
            pltpu.VMEM((CHUNK,), jnp.int32),
            pltpu.VMEM((CHUNK,), jnp.int32),
            pltpu.VMEM((CHUNK, H), jnp.float32),
            pltpu.VMEM_SHARED((N, H), jnp.float32),
            pltpu.SemaphoreType.DMA,
        ],
    )
    def sc_scatter(m_hbm, gidx_hbm, sidx_hbm, zeros_hbm, out_hbm,
                   gi_v, si_v, rows_v, acc, sem):
        c = lax.axis_index("c")
        s = lax.axis_index("s")
        wid = s * NC + c

        pltpu.sync_copy(zeros_hbm, acc.at[pl.ds(s * STRIPE, STRIPE)])

        @pl.when(s == NS - 1)
        def _zero_tail():
            pltpu.sync_copy(zeros_hbm.at[pl.ds(0, TAIL)],
                            acc.at[pl.ds(NS * STRIPE, TAIL)])

        plsc.subcore_barrier()

        def body(g, carry):
            base = wid * EPW + g * CHUNK
            pltpu.sync_copy(gidx_hbm.at[pl.ds(base, CHUNK)], gi_v)
            pltpu.sync_copy(sidx_hbm.at[pl.ds(base, CHUNK)], si_v)
            pltpu.async_copy(m_hbm.at[gi_v], rows_v, sem).wait()
            pltpu.sync_copy(rows_v, acc.at[si_v], add=True)
            return carry

        lax.fori_loop(0, NCHUNK, body, 0)
        plsc.subcore_barrier()

        pltpu.sync_copy(acc.at[pl.ds(s * STRIPE, STRIPE)],
                        out_hbm.at[pl.ds(c * N + s * STRIPE, STRIPE)])

        @pl.when(s == NS - 1)
        def _copy_tail():
            pltpu.sync_copy(acc.at[pl.ds(NS * STRIPE, TAIL)],
                            out_hbm.at[pl.ds(c * N + NS * STRIPE, TAIL)])

    return sc_scatter


_sc_scatter = _sc_scatter_make()


R = 400
G3 = 3 * H


def _rowsum(v):
    acc = v[:, 0:8] * 0.0
    for j in range(16):
        acc = acc + v[:, j * 8:(j + 1) * 8]
    a4 = acc[:, 0:4] + acc[:, 4:8]
    a2 = a4[:, 0:2] + a4[:, 2:4]
    return a2[:, 0:1] + a2[:, 1:2]


def _lnorm(hn, lng, lnb):
    mu = _rowsum(hn) * (1.0 / H)
    var = _rowsum((hn - mu) ** 2) * (1.0 / H)
    return (hn - mu) / jnp.sqrt(var + 1e-5) * lng + lnb


def _tc_step_body(s0_ref, s1_ref, x_ref, h_ref, wih_ref, whh_ref,
                  bih_ref, bhh_ref, lng_ref, lnb_ref, wn_ref, bn_ref,
                  h_out_ref, m_out_ref):
    msg = s0_ref[...] + s1_ref[...]
    x = x_ref[...]
    h = h_ref[...]
    xin = jnp.concatenate([msg, x], axis=-1)
    gi = jnp.dot(xin, wih_ref[...]) + bih_ref[...]
    gh = jnp.dot(h, whh_ref[...]) + bhh_ref[...]
    r = jax.nn.sigmoid(gi[:, :H] + gh[:, :H])
    z = jax.nn.sigmoid(gi[:, H:2 * H] + gh[:, H:2 * H])
    n = jnp.tanh(gi[:, 2 * H:] + r * gh[:, 2 * H:])
    hn = (1.0 - z) * n + z * h
    hn = _lnorm(hn, lng_ref[...], lnb_ref[...])
    h_out_ref[...] = hn
    m_out_ref[...] = jnp.dot(hn, wn_ref[...]) + bn_ref[...]


def _tc_step(s0, s1, x, h, wiht, whht, bih, bhh, lng, lnb, wnt, bn):
    row = lambda i: (i, 0)
    full = lambda i: (0, 0)
    return pl.pallas_call(
        _tc_step_body,
        grid=(N // R,),
        in_specs=[
            pl.BlockSpec((R, H), row),
            pl.BlockSpec((R, H), row),
            pl.BlockSpec((R, F), row),
            pl.BlockSpec((R, H), row),
            pl.BlockSpec((H + F, G3), full),
            pl.BlockSpec((H, G3), full),
            pl.BlockSpec((1, G3), full),
            pl.BlockSpec((1, G3), full),
            pl.BlockSpec((1, H), full),
            pl.BlockSpec((1, H), full),
            pl.BlockSpec((H, H), full),
            pl.BlockSpec((1, H), full),
        ],
        out_specs=[
            pl.BlockSpec((R, H), row),
            pl.BlockSpec((R, H), row),
        ],
        out_shape=[
            jax.ShapeDtypeStruct((N, H), jnp.float32),
            jax.ShapeDtypeStruct((N, H), jnp.float32),
        ],
    )(s0, s1, x, h, wiht, whht, bih, bhh, lng, lnb, wnt, bn)


def _tc_step1_body(s0_ref, s1_ref, x_ref, wih_ref, gh_ref, bih_ref,
                   lng_ref, lnb_ref, wn_ref, bn_ref, h_out_ref, m_out_ref):
    msg = s0_ref[...] + s1_ref[...]
    x = x_ref[...]
    xin = jnp.concatenate([msg, x], axis=-1)
    gi = jnp.dot(xin, wih_ref[...]) + bih_ref[...]
    gh = gh_ref[...]
    r = jax.nn.sigmoid(gi[:, :H] + gh[:, :H])
    z = jax.nn.sigmoid(gi[:, H:2 * H] + gh[:, H:2 * H])
    n = jnp.tanh(gi[:, 2 * H:] + r * gh[:, 2 * H:])
    hn = (1.0 - z) * n + z
    hn = _lnorm(hn, lng_ref[...], lnb_ref[...])
    h_out_ref[...] = hn
    m_out_ref[...] = jnp.dot(hn, wn_ref[...]) + bn_ref[...]


def _tc_step1(s0, s1, x, wiht, gh_row, bih, lng, lnb, wnt, bn):
    row = lambda i: (i, 0)
    full = lambda i: (0, 0)
    return pl.pallas_call(
        _tc_step1_body,
        grid=(N // R,),
        in_specs=[
            pl.BlockSpec((R, H), row),
            pl.BlockSpec((R, H), row),
            pl.BlockSpec((R, F), row),
            pl.BlockSpec((H + F, G3), full),
            pl.BlockSpec((1, G3), full),
            pl.BlockSpec((1, G3), full),
            pl.BlockSpec((1, H), full),
            pl.BlockSpec((1, H), full),
            pl.BlockSpec((H, H), full),
            pl.BlockSpec((1, H), full),
        ],
        out_specs=[
            pl.BlockSpec((R, H), row),
            pl.BlockSpec((R, H), row),
        ],
        out_shape=[
            jax.ShapeDtypeStruct((N, H), jnp.float32),
            jax.ShapeDtypeStruct((N, H), jnp.float32),
        ],
    )(s0, s1, x, wiht, gh_row, bih, lng, lnb, wnt, bn)


def kernel(x, edge_index, W_agg, b_agg, W_agg_r, b_agg_r, Wih, Whh, bih, bhh,
           Wih_r, Whh_r, bih_r, bhh_r, ln_g, ln_b):
    src = edge_index[0]
    dst = edge_index[1]
    zeros = jnp.zeros((STRIPE, H), jnp.float32)

    perm_f = jnp.argsort(dst, stable=True)
    sg_f, ds_f = src[perm_f], dst[perm_f]
    perm_r = jnp.argsort(src, stable=True)
    sg_r, ds_r = dst[perm_r], src[perm_r]

    wiht_f, wiht_r = Wih.T, Wih_r.T
    whht_f, whht_r = Whh.T, Whh_r.T
    wnt_f, wnt_r = W_agg.T, W_agg_r.T
    bih_f, bhh_f = bih.reshape(1, G3), bhh.reshape(1, G3)
    bih_r2, bhh_r2 = bih_r.reshape(1, G3), bhh_r.reshape(1, G3)
    lng, lnb = ln_g.reshape(1, H), ln_b.reshape(1, H)
    bn_f, bn_r = b_agg.reshape(1, H), b_agg_r.reshape(1, H)

    m = jnp.broadcast_to(W_agg.sum(axis=1) + b_agg, (N, H))
    gh_row = (Whh.sum(axis=1) + bhh).reshape(1, G3)

    p = _sc_scatter(m, sg_f, ds_f, zeros)
    h, m = _tc_step1(p[:N], p[N:], x, wiht_f, gh_row,
                     bih_f, lng, lnb, wnt_r, bn_r)
    p = _sc_scatter(m, sg_r, ds_r, zeros)
    h, m = _tc_step(p[:N], p[N:], x, h, wiht_r, whht_r,
                    bih_r2, bhh_r2, lng, lnb, wnt_f, bn_f)
    for _ in range(ROUNDS - 1):
        p = _sc_scatter(m, sg_f, ds_f, zeros)
        h, m = _tc_step(p[:N], p[N:], x, h, wiht_f, whht_f,
                        bih_f, bhh_f, lng, lnb, wnt_r, bn_r)
        p = _sc_scatter(m, sg_r, ds_r, zeros)
        h, m = _tc_step(p[:N], p[N:], x, h, wiht_r, whht_r,
                        bih_r2, bhh_r2, lng, lnb, wnt_f, bn_f)
    return h


# speedup vs baseline: 4.3478x; 1.3926x over previous
import functools

import jax
import jax.numpy as jnp
from jax import lax
from jax.experimental import pallas as pl
from jax.experimental.pallas import tpu as pltpu
from jax.experimental.pallas import tpu_sc as plsc

N = 10000
E = 320000
H = 128
F = 128
ROUNDS = 3

NC = 2
NS = 16
NW = NC * NS
EPW = E // NW
CHUNK = 80
NCHUNK = EPW // CHUNK
STRIPE = 624
TAIL = N - NS * STRIPE
ZROWS = 104


def _sc_scatter_make():
    mesh = plsc.VectorSubcoreMesh(core_axis_name="c", subcore_axis_name="s")

    @functools.partial(
        pl.kernel,
        out_type=jax.ShapeDtypeStruct((NC * N, H), jnp.float32),
        name="edge_scatter_add",
        mesh=mesh,
        scratch_types=[
            pltpu.VMEM((EPW,), jnp.int32),
            pltpu.VMEM((NCHUNK, CHUNK), jnp.int32),
            pltpu.VMEM((CHUNK, H), jnp.float32),
            pltpu.VMEM((CHUNK, H), jnp.float32),
            pltpu.VMEM_SHARED((N, H), jnp.float32),
            pltpu.SemaphoreType.DMA,
            pltpu.SemaphoreType.DMA,
        ],
    )
    def sc_scatter(m_hbm, gidx_hbm, sidx_hbm, zeros_hbm, out_hbm,
                   gi_v, si_v, rows0, rows1, acc, sem0, sem1):
        c = lax.axis_index("c")
        s = lax.axis_index("s")
        wid = s * NC + c

        pltpu.sync_copy(gidx_hbm.at[wid], gi_v)
        pltpu.sync_copy(sidx_hbm.at[wid], si_v)

        pltpu.sync_copy(zeros_hbm, acc.at[pl.ds(s * STRIPE, STRIPE)])

        @pl.when(s == NS - 1)
        def _zero_tail():
            pltpu.sync_copy(zeros_hbm.at[pl.ds(0, TAIL)],
                            acc.at[pl.ds(NS * STRIPE, TAIL)])

        plsc.subcore_barrier()

        bufs = (rows0, rows1)
        sems = (sem0, sem1)

        def start(g, buf, sem):
            pltpu.async_copy(m_hbm.at[gi_v.at[pl.ds(g * CHUNK, CHUNK)]],
                             buf, sem)

        def finish(g, buf, sem):
            pltpu.make_async_copy(m_hbm.at[gi_v.at[pl.ds(g * CHUNK, CHUNK)]],
                                  buf, sem).wait()
            pltpu.sync_copy(buf, acc.at[si_v.at[g]], add=True)

        start(0, bufs[0], sems[0])

        def body(i, carry):
            for b in range(2):
                g = 2 * i + b
                start(g + 1, bufs[1 - b], sems[1 - b])
                finish(g, bufs[b], sems[b])
            return carry

        lax.fori_loop(0, (NCHUNK - 1) // 2, body, 0)
        finish(NCHUNK - 1, bufs[0], sems[0])
        plsc.subcore_barrier()

        pltpu.sync_copy(acc.at[pl.ds(s * STRIPE, STRIPE)],
                        out_hbm.at[pl.ds(c * N + s * STRIPE, STRIPE)])

        @pl.when(s == NS - 1)
        def _copy_tail():
            pltpu.sync_copy(acc.at[pl.ds(NS * STRIPE, TAIL)],
                            out_hbm.at[pl.ds(c * N + NS * STRIPE, TAIL)])

    return sc_scatter


_sc_scatter = _sc_scatter_make()


R = 400
G3 = 3 * H


def _rowsum(v):
    acc = v[:, 0:8] * 0.0
    for j in range(16):
        acc = acc + v[:, j * 8:(j + 1) * 8]
    a4 = acc[:, 0:4] + acc[:, 4:8]
    a2 = a4[:, 0:2] + a4[:, 2:4]
    return a2[:, 0:1] + a2[:, 1:2]


def _lnorm(hn, lng, lnb):
    mu = _rowsum(hn) * (1.0 / H)
    var = _rowsum((hn - mu) ** 2) * (1.0 / H)
    return (hn - mu) / jnp.sqrt(var + 1e-5) * lng + lnb


def _tc_step_body(s0_ref, s1_ref, x_ref, h_ref, wih_ref, whh_ref,
                  bih_ref, bhh_ref, lng_ref, lnb_ref, wn_ref, bn_ref,
                  h_out_ref, m_out_ref):
    msg = s0_ref[...] + s1_ref[...]
    x = x_ref[...]
    h = h_ref[...]
    xin = jnp.concatenate([msg, x], axis=-1)
    gi = jnp.dot(xin, wih_ref[...]) + bih_ref[...]
    gh = jnp.dot(h, whh_ref[...]) + bhh_ref[...]
    r = jax.nn.sigmoid(gi[:, :H] + gh[:, :H])
    z = jax.nn.sigmoid(gi[:, H:2 * H] + gh[:, H:2 * H])
    n = jnp.tanh(gi[:, 2 * H:] + r * gh[:, 2 * H:])
    hn = (1.0 - z) * n + z * h
    hn = _lnorm(hn, lng_ref[...], lnb_ref[...])
    h_out_ref[...] = hn
    m_out_ref[...] = jnp.dot(hn, wn_ref[...]) + bn_ref[...]


def _tc_step(s0, s1, x, h, wiht, whht, bih, bhh, lng, lnb, wnt, bn):
    row = lambda i: (i, 0)
    full = lambda i: (0, 0)
    return pl.pallas_call(
        _tc_step_body,
        grid=(N // R,),
        in_specs=[
            pl.BlockSpec((R, H), row),
            pl.BlockSpec((R, H), row),
            pl.BlockSpec((R, F), row),
            pl.BlockSpec((R, H), row),
            pl.BlockSpec((H + F, G3), full),
            pl.BlockSpec((H, G3), full),
            pl.BlockSpec((1, G3), full),
            pl.BlockSpec((1, G3), full),
            pl.BlockSpec((1, H), full),
            pl.BlockSpec((1, H), full),
            pl.BlockSpec((H, H), full),
            pl.BlockSpec((1, H), full),
        ],
        out_specs=[
            pl.BlockSpec((R, H), row),
            pl.BlockSpec((R, H), row),
        ],
        out_shape=[
            jax.ShapeDtypeStruct((N, H), jnp.float32),
            jax.ShapeDtypeStruct((N, H), jnp.float32),
        ],
    )(s0, s1, x, h, wiht, whht, bih, bhh, lng, lnb, wnt, bn)


def _tc_step1_body(s0_ref, s1_ref, x_ref, wih_ref, gh_ref, bih_ref,
                   lng_ref, lnb_ref, wn_ref, bn_ref, h_out_ref, m_out_ref):
    msg = s0_ref[...] + s1_ref[...]
    x = x_ref[...]
    xin = jnp.concatenate([msg, x], axis=-1)
    gi = jnp.dot(xin, wih_ref[...]) + bih_ref[...]
    gh = gh_ref[...]
    r = jax.nn.sigmoid(gi[:, :H] + gh[:, :H])
    z = jax.nn.sigmoid(gi[:, H:2 * H] + gh[:, H:2 * H])
    n = jnp.tanh(gi[:, 2 * H:] + r * gh[:, 2 * H:])
    hn = (1.0 - z) * n + z
    hn = _lnorm(hn, lng_ref[...], lnb_ref[...])
    h_out_ref[...] = hn
    m_out_ref[...] = jnp.dot(hn, wn_ref[...]) + bn_ref[...]


def _tc_step1(s0, s1, x, wiht, gh_row, bih, lng, lnb, wnt, bn):
    row = lambda i: (i, 0)
    full = lambda i: (0, 0)
    return pl.pallas_call(
        _tc_step1_body,
        grid=(N // R,),
        in_specs=[
            pl.BlockSpec((R, H), row),
            pl.BlockSpec((R, H), row),
            pl.BlockSpec((R, F), row),
            pl.BlockSpec((H + F, G3), full),
            pl.BlockSpec((1, G3), full),
            pl.BlockSpec((1, G3), full),
            pl.BlockSpec((1, H), full),
            pl.BlockSpec((1, H), full),
            pl.BlockSpec((H, H), full),
            pl.BlockSpec((1, H), full),
        ],
        out_specs=[
            pl.BlockSpec((R, H), row),
            pl.BlockSpec((R, H), row),
        ],
        out_shape=[
            jax.ShapeDtypeStruct((N, H), jnp.float32),
            jax.ShapeDtypeStruct((N, H), jnp.float32),
        ],
    )(s0, s1, x, wiht, gh_row, bih, lng, lnb, wnt, bn)


def kernel(x, edge_index, W_agg, b_agg, W_agg_r, b_agg_r, Wih, Whh, bih, bhh,
           Wih_r, Whh_r, bih_r, bhh_r, ln_g, ln_b):
    src = edge_index[0]
    dst = edge_index[1]
    zeros = jnp.zeros((STRIPE, H), jnp.float32)

    gshp = (NW, EPW)
    sshp = (NW, NCHUNK, CHUNK)
    perm_f = jnp.argsort(dst, stable=True)
    sg_f, ds_f = src[perm_f].reshape(gshp), dst[perm_f].reshape(sshp)
    perm_r = jnp.argsort(src, stable=True)
    sg_r, ds_r = dst[perm_r].reshape(gshp), src[perm_r].reshape(sshp)

    wiht_f, wiht_r = Wih.T, Wih_r.T
    whht_f, whht_r = Whh.T, Whh_r.T
    wnt_f, wnt_r = W_agg.T, W_agg_r.T
    bih_f, bhh_f = bih.reshape(1, G3), bhh.reshape(1, G3)
    bih_r2, bhh_r2 = bih_r.reshape(1, G3), bhh_r.reshape(1, G3)
    lng, lnb = ln_g.reshape(1, H), ln_b.reshape(1, H)
    bn_f, bn_r = b_agg.reshape(1, H), b_agg_r.reshape(1, H)

    m = jnp.broadcast_to(W_agg.sum(axis=1) + b_agg, (N, H))
    gh_row = (Whh.sum(axis=1) + bhh).reshape(1, G3)

    p = _sc_scatter(m, sg_f, ds_f, zeros)
    h, m = _tc_step1(p[:N], p[N:], x, wiht_f, gh_row,
                     bih_f, lng, lnb, wnt_r, bn_r)
    p = _sc_scatter(m, sg_r, ds_r, zeros)
    h, m = _tc_step(p[:N], p[N:], x, h, wiht_r, whht_r,
                    bih_r2, bhh_r2, lng, lnb, wnt_f, bn_f)
    for _ in range(ROUNDS - 1):
        p = _sc_scatter(m, sg_f, ds_f, zeros)
        h, m = _tc_step(p[:N], p[N:], x, h, wiht_f, whht_f,
                        bih_f, bhh_f, lng, lnb, wnt_r, bn_r)
        p = _sc_scatter(m, sg_r, ds_r, zeros)
        h, m = _tc_step(p[:N], p[N:], x, h, wiht_r, whht_r,
                        bih_r2, bhh_r2, lng, lnb, wnt_f, bn_f)
    return h


# speedup vs baseline: 4.7127x; 1.0839x over previous
import functools

import jax
import jax.numpy as jnp
from jax import lax
from jax.experimental import pallas as pl
from jax.experimental.pallas import tpu as pltpu
from jax.experimental.pallas import tpu_sc as plsc

N = 10000
E = 320000
H = 128
F = 128
ROUNDS = 3

NC = 2
NS = 16
NW = NC * NS
EPW = E // NW
CHUNK = 80
NCHUNK = EPW // CHUNK
STRIPE = 624
TAIL = N - NS * STRIPE
ZROWS = 104


def _sc_scatter_make():
    mesh = plsc.VectorSubcoreMesh(core_axis_name="c", subcore_axis_name="s")

    @functools.partial(
        pl.kernel,
        out_type=jax.ShapeDtypeStruct((NC * N, H), jnp.float32),
        name="edge_scatter_add",
        mesh=mesh,
        scratch_types=[
            pltpu.VMEM((EPW,), jnp.int32),
            pltpu.VMEM((NCHUNK, CHUNK), jnp.int32),
            pltpu.VMEM((CHUNK, H), jnp.float32),
            pltpu.VMEM((CHUNK, H), jnp.float32),
            pltpu.VMEM_SHARED((N, H), jnp.float32),
            pltpu.SemaphoreType.DMA,
            pltpu.SemaphoreType.DMA,
        ],
    )
    def sc_scatter(m_hbm, gidx_hbm, sidx_hbm, zeros_hbm, out_hbm,
                   gi_v, si_v, rows0, rows1, acc, sem0, sem1):
        c = lax.axis_index("c")
        s = lax.axis_index("s")
        wid = s * NC + c

        pltpu.sync_copy(gidx_hbm.at[wid], gi_v)
        pltpu.sync_copy(sidx_hbm.at[wid], si_v)

        pltpu.sync_copy(zeros_hbm, acc.at[pl.ds(s * STRIPE, STRIPE)])

        @pl.when(s == NS - 1)
        def _zero_tail():
            pltpu.sync_copy(zeros_hbm.at[pl.ds(0, TAIL)],
                            acc.at[pl.ds(NS * STRIPE, TAIL)])

        plsc.subcore_barrier()

        bufs = (rows0, rows1)
        sems = (sem0, sem1)

        def start(g, buf, sem):
            pltpu.async_copy(m_hbm.at[gi_v.at[pl.ds(g * CHUNK, CHUNK)]],
                             buf, sem)

        def finish(g, buf, sem):
            pltpu.make_async_copy(m_hbm.at[gi_v.at[pl.ds(g * CHUNK, CHUNK)]],
                                  buf, sem).wait()
            pltpu.sync_copy(buf, acc.at[si_v.at[g]], add=True)

        start(0, bufs[0], sems[0])

        def body(i, carry):
            for b in range(2):
                g = 2 * i + b
                start(g + 1, bufs[1 - b], sems[1 - b])
                finish(g, bufs[b], sems[b])
            return carry

        lax.fori_loop(0, (NCHUNK - 1) // 2, body, 0)
        finish(NCHUNK - 1, bufs[0], sems[0])
        plsc.subcore_barrier()

        pltpu.sync_copy(acc.at[pl.ds(s * STRIPE, STRIPE)],
                        out_hbm.at[pl.ds(c * N + s * STRIPE, STRIPE)])

        @pl.when(s == NS - 1)
        def _copy_tail():
            pltpu.sync_copy(acc.at[pl.ds(NS * STRIPE, TAIL)],
                            out_hbm.at[pl.ds(c * N + NS * STRIPE, TAIL)])

    return sc_scatter


_sc_scatter = _sc_scatter_make()


R = 400
G3 = 3 * H


def _rowsum(v):
    acc = v[:, 0:8] * 0.0
    for j in range(16):
        acc = acc + v[:, j * 8:(j + 1) * 8]
    a4 = acc[:, 0:4] + acc[:, 4:8]
    a2 = a4[:, 0:2] + a4[:, 2:4]
    return a2[:, 0:1] + a2[:, 1:2]


def _lnorm(hn, lng, lnb):
    mu = _rowsum(hn) * (1.0 / H)
    var = _rowsum((hn - mu) ** 2) * (1.0 / H)
    return (hn - mu) / jnp.sqrt(var + 1e-5) * lng + lnb


def _tc_step_body(s0_ref, s1_ref, x_ref, h_ref, wih_ref, whh_ref,
                  bih_ref, bhh_ref, lng_ref, lnb_ref, wn_ref, bn_ref,
                  h_out_ref, m_out_ref):
    msg = s0_ref[...] + s1_ref[...]
    x = x_ref[...]
    h = h_ref[...]
    xin = jnp.concatenate([msg, x], axis=-1)
    gi = jnp.dot(xin, wih_ref[...]) + bih_ref[...]
    gh = jnp.dot(h, whh_ref[...]) + bhh_ref[...]
    r = jax.nn.sigmoid(gi[:, :H] + gh[:, :H])
    z = jax.nn.sigmoid(gi[:, H:2 * H] + gh[:, H:2 * H])
    n = jnp.tanh(gi[:, 2 * H:] + r * gh[:, 2 * H:])
    hn = (1.0 - z) * n + z * h
    hn = _lnorm(hn, lng_ref[...], lnb_ref[...])
    h_out_ref[...] = hn
    m_out_ref[...] = jnp.dot(hn, wn_ref[...]) + bn_ref[...]


def _tc_step(s0, s1, x, h, wiht, whht, bih, bhh, lng, lnb, wnt, bn):
    row = lambda i: (i, 0)
    full = lambda i: (0, 0)
    return pl.pallas_call(
        _tc_step_body,
        grid=(N // R,),
        in_specs=[
            pl.BlockSpec((R, H), row),
            pl.BlockSpec((R, H), row),
            pl.BlockSpec((R, F), row),
            pl.BlockSpec((R, H), row),
            pl.BlockSpec((H + F, G3), full),
            pl.BlockSpec((H, G3), full),
            pl.BlockSpec((1, G3), full),
            pl.BlockSpec((1, G3), full),
            pl.BlockSpec((1, H), full),
            pl.BlockSpec((1, H), full),
            pl.BlockSpec((H, H), full),
            pl.BlockSpec((1, H), full),
        ],
        out_specs=[
            pl.BlockSpec((R, H), row),
            pl.BlockSpec((R, H), row),
        ],
        out_shape=[
            jax.ShapeDtypeStruct((N, H), jnp.float32),
            jax.ShapeDtypeStruct((N, H), jnp.float32),
        ],
    )(s0, s1, x, h, wiht, whht, bih, bhh, lng, lnb, wnt, bn)


def _tc_step1_body(s0_ref, s1_ref, x_ref, wih_ref, gh_ref, bih_ref,
                   lng_ref, lnb_ref, wn_ref, bn_ref, h_out_ref, m_out_ref):
    msg = s0_ref[...] + s1_ref[...]
    x = x_ref[...]
    xin = jnp.concatenate([msg, x], axis=-1)
    gi = jnp.dot(xin, wih_ref[...]) + bih_ref[...]
    gh = gh_ref[...]
    r = jax.nn.sigmoid(gi[:, :H] + gh[:, :H])
    z = jax.nn.sigmoid(gi[:, H:2 * H] + gh[:, H:2 * H])
    n = jnp.tanh(gi[:, 2 * H:] + r * gh[:, 2 * H:])
    hn = (1.0 - z) * n + z
    hn = _lnorm(hn, lng_ref[...], lnb_ref[...])
    h_out_ref[...] = hn
    m_out_ref[...] = jnp.dot(hn, wn_ref[...]) + bn_ref[...]


def _tc_step1(s0, s1, x, wiht, gh_row, bih, lng, lnb, wnt, bn):
    row = lambda i: (i, 0)
    full = lambda i: (0, 0)
    return pl.pallas_call(
        _tc_step1_body,
        grid=(N // R,),
        in_specs=[
            pl.BlockSpec((R, H), row),
            pl.BlockSpec((R, H), row),
            pl.BlockSpec((R, F), row),
            pl.BlockSpec((H + F, G3), full),
            pl.BlockSpec((1, G3), full),
            pl.BlockSpec((1, G3), full),
            pl.BlockSpec((1, H), full),
            pl.BlockSpec((1, H), full),
            pl.BlockSpec((H, H), full),
            pl.BlockSpec((1, H), full),
        ],
        out_specs=[
            pl.BlockSpec((R, H), row),
            pl.BlockSpec((R, H), row),
        ],
        out_shape=[
            jax.ShapeDtypeStruct((N, H), jnp.float32),
            jax.ShapeDtypeStruct((N, H), jnp.float32),
        ],
    )(s0, s1, x, wiht, gh_row, bih, lng, lnb, wnt, bn)


def kernel(x, edge_index, W_agg, b_agg, W_agg_r, b_agg_r, Wih, Whh, bih, bhh,
           Wih_r, Whh_r, bih_r, bhh_r, ln_g, ln_b):
    src = edge_index[0]
    dst = edge_index[1]
    zeros = jnp.zeros((STRIPE, H), jnp.float32)

    gshp = (NW, EPW)
    sshp = (NW, NCHUNK, CHUNK)
    ds_f, sg_f = lax.sort([dst, src], num_keys=1, is_stable=True)
    ds_r, sg_r = lax.sort([src, dst], num_keys=1, is_stable=True)
    sg_f, ds_f = sg_f.reshape(gshp), ds_f.reshape(sshp)
    sg_r, ds_r = sg_r.reshape(gshp), ds_r.reshape(sshp)

    wiht_f, wiht_r = Wih.T, Wih_r.T
    whht_f, whht_r = Whh.T, Whh_r.T
    wnt_f, wnt_r = W_agg.T, W_agg_r.T
    bih_f, bhh_f = bih.reshape(1, G3), bhh.reshape(1, G3)
    bih_r2, bhh_r2 = bih_r.reshape(1, G3), bhh_r.reshape(1, G3)
    lng, lnb = ln_g.reshape(1, H), ln_b.reshape(1, H)
    bn_f, bn_r = b_agg.reshape(1, H), b_agg_r.reshape(1, H)

    m = jnp.broadcast_to(W_agg.sum(axis=1) + b_agg, (N, H))
    gh_row = (Whh.sum(axis=1) + bhh).reshape(1, G3)

    p = _sc_scatter(m, sg_f, ds_f, zeros)
    h, m = _tc_step1(p[:N], p[N:], x, wiht_f, gh_row,
                     bih_f, lng, lnb, wnt_r, bn_r)
    p = _sc_scatter(m, sg_r, ds_r, zeros)
    h, m = _tc_step(p[:N], p[N:], x, h, wiht_r, whht_r,
                    bih_r2, bhh_r2, lng, lnb, wnt_f, bn_f)
    for _ in range(ROUNDS - 1):
        p = _sc_scatter(m, sg_f, ds_f, zeros)
        h, m = _tc_step(p[:N], p[N:], x, h, wiht_f, whht_f,
                        bih_f, bhh_f, lng, lnb, wnt_r, bn_r)
        p = _sc_scatter(m, sg_r, ds_r, zeros)
        h, m = _tc_step(p[:N], p[N:], x, h, wiht_r, whht_r,
                        bih_r2, bhh_r2, lng, lnb, wnt_f, bn_f)
    return h


# speedup vs baseline: 5.4012x; 1.1461x over previous
import functools

import jax
import jax.numpy as jnp
from jax import lax
from jax.experimental import pallas as pl
from jax.experimental.pallas import tpu as pltpu
from jax.experimental.pallas import tpu_sc as plsc

N = 10000
E = 320000
H = 128
F = 128
ROUNDS = 3

NC = 2
NS = 16
NW = NC * NS
EPW = E // NW
CHUNK = 80
NCHUNK = EPW // CHUNK
STRIPE = 624
TAIL = N - NS * STRIPE
ZROWS = 104


def _sc_scatter_make():
    mesh = plsc.VectorSubcoreMesh(core_axis_name="c", subcore_axis_name="s")

    @functools.partial(
        pl.kernel,
        out_type=jax.ShapeDtypeStruct((NC * N, H), jnp.float32),
        name="edge_scatter_add",
        mesh=mesh,
        scratch_types=[
            pltpu.VMEM((EPW,), jnp.int32),
            pltpu.VMEM((NCHUNK, CHUNK), jnp.int32),
            pltpu.VMEM((CHUNK, H), jnp.float32),
            pltpu.VMEM((CHUNK, H), jnp.float32),
            pltpu.VMEM_SHARED((N, H), jnp.float32),
            pltpu.SemaphoreType.DMA,
            pltpu.SemaphoreType.DMA,
        ],
    )
    def sc_scatter(m_hbm, gidx_hbm, sidx_hbm, zeros_hbm, out_hbm,
                   gi_v, si_v, rows0, rows1, acc, sem0, sem1):
        c = lax.axis_index("c")
        s = lax.axis_index("s")
        wid = s * NC + c

        pltpu.sync_copy(gidx_hbm.at[wid], gi_v)
        pltpu.sync_copy(sidx_hbm.at[wid], si_v)

        pltpu.sync_copy(zeros_hbm, acc.at[pl.ds(s * STRIPE, STRIPE)])

        @pl.when(s == NS - 1)
        def _zero_tail():
            pltpu.sync_copy(zeros_hbm.at[pl.ds(0, TAIL)],
                            acc.at[pl.ds(NS * STRIPE, TAIL)])

        plsc.subcore_barrier()

        bufs = (rows0, rows1)
        sems = (sem0, sem1)

        def start(g, buf, sem):
            pltpu.async_copy(m_hbm.at[gi_v.at[pl.ds(g * CHUNK, CHUNK)]],
                             buf, sem)

        def finish(g, buf, sem):
            pltpu.make_async_copy(m_hbm.at[gi_v.at[pl.ds(g * CHUNK, CHUNK)]],
                                  buf, sem).wait()
            pltpu.sync_copy(buf, acc.at[si_v.at[g]], add=True)

        start(0, bufs[0], sems[0])

        def body(i, carry):
            for b in range(2):
                g = 2 * i + b
                start(g + 1, bufs[1 - b], sems[1 - b])
                finish(g, bufs[b], sems[b])
            return carry

        lax.fori_loop(0, (NCHUNK - 1) // 2, body, 0)
        finish(NCHUNK - 1, bufs[0], sems[0])
        plsc.subcore_barrier()

        pltpu.sync_copy(acc.at[pl.ds(s * STRIPE, STRIPE)],
                        out_hbm.at[pl.ds(c * N + s * STRIPE, STRIPE)])

        @pl.when(s == NS - 1)
        def _copy_tail():
            pltpu.sync_copy(acc.at[pl.ds(NS * STRIPE, TAIL)],
                            out_hbm.at[pl.ds(c * N + NS * STRIPE, TAIL)])

    return sc_scatter


_sc_scatter = _sc_scatter_make()


R = 400
G3 = 3 * H


def _rowsum(v):
    acc = v[:, 0:8] * 0.0
    for j in range(16):
        acc = acc + v[:, j * 8:(j + 1) * 8]
    a4 = acc[:, 0:4] + acc[:, 4:8]
    a2 = a4[:, 0:2] + a4[:, 2:4]
    return a2[:, 0:1] + a2[:, 1:2]


def _lnorm(hn, lng, lnb):
    mu = _rowsum(hn) * (1.0 / H)
    var = _rowsum((hn - mu) ** 2) * (1.0 / H)
    return (hn - mu) / jnp.sqrt(var + 1e-5) * lng + lnb


def _tc_step_body(s0_ref, s1_ref, x_ref, h_ref, wih_ref, whh_ref,
                  bih_ref, bhh_ref, lng_ref, lnb_ref, wn_ref, bn_ref,
                  h_out_ref, m_out_ref):
    msg = s0_ref[...] + s1_ref[...]
    x = x_ref[...]
    h = h_ref[...]
    xin = jnp.concatenate([msg, x], axis=-1)
    gi = jnp.dot(xin, wih_ref[...]) + bih_ref[...]
    gh = jnp.dot(h, whh_ref[...]) + bhh_ref[...]
    r = jax.nn.sigmoid(gi[:, :H] + gh[:, :H])
    z = jax.nn.sigmoid(gi[:, H:2 * H] + gh[:, H:2 * H])
    n = jnp.tanh(gi[:, 2 * H:] + r * gh[:, 2 * H:])
    hn = (1.0 - z) * n + z * h
    hn = _lnorm(hn, lng_ref[...], lnb_ref[...])
    h_out_ref[...] = hn
    m_out_ref[...] = jnp.dot(hn, wn_ref[...]) + bn_ref[...]


def _tc_step(s0, s1, x, h, wiht, whht, bih, bhh, lng, lnb, wnt, bn):
    row = lambda i: (i, 0)
    full = lambda i: (0, 0)
    return pl.pallas_call(
        _tc_step_body,
        grid=(N // R,),
        in_specs=[
            pl.BlockSpec((R, H), row),
            pl.BlockSpec((R, H), row),
            pl.BlockSpec((R, F), row),
            pl.BlockSpec((R, H), row),
            pl.BlockSpec((H + F, G3), full),
            pl.BlockSpec((H, G3), full),
            pl.BlockSpec((1, G3), full),
            pl.BlockSpec((1, G3), full),
            pl.BlockSpec((1, H), full),
            pl.BlockSpec((1, H), full),
            pl.BlockSpec((H, H), full),
            pl.BlockSpec((1, H), full),
        ],
        out_specs=[
            pl.BlockSpec((R, H), row),
            pl.BlockSpec((R, H), row),
        ],
        out_shape=[
            jax.ShapeDtypeStruct((N, H), jnp.float32),
            jax.ShapeDtypeStruct((N, H), jnp.float32),
        ],
    )(s0, s1, x, h, wiht, whht, bih, bhh, lng, lnb, wnt, bn)


def _tc_step1_body(s0_ref, s1_ref, x_ref, wih_ref, gh_ref, bih_ref,
                   lng_ref, lnb_ref, wn_ref, bn_ref, h_out_ref, m_out_ref):
    msg = s0_ref[...] + s1_ref[...]
    x = x_ref[...]
    xin = jnp.concatenate([msg, x], axis=-1)
    gi = jnp.dot(xin, wih_ref[...]) + bih_ref[...]
    gh = gh_ref[...]
    r = jax.nn.sigmoid(gi[:, :H] + gh[:, :H])
    z = jax.nn.sigmoid(gi[:, H:2 * H] + gh[:, H:2 * H])
    n = jnp.tanh(gi[:, 2 * H:] + r * gh[:, 2 * H:])
    hn = (1.0 - z) * n + z
    hn = _lnorm(hn, lng_ref[...], lnb_ref[...])
    h_out_ref[...] = hn
    m_out_ref[...] = jnp.dot(hn, wn_ref[...]) + bn_ref[...]


def _tc_step1(s0, s1, x, wiht, gh_row, bih, lng, lnb, wnt, bn):
    row = lambda i: (i, 0)
    full = lambda i: (0, 0)
    return pl.pallas_call(
        _tc_step1_body,
        grid=(N // R,),
        in_specs=[
            pl.BlockSpec((R, H), row),
            pl.BlockSpec((R, H), row),
            pl.BlockSpec((R, F), row),
            pl.BlockSpec((H + F, G3), full),
            pl.BlockSpec((1, G3), full),
            pl.BlockSpec((1, G3), full),
            pl.BlockSpec((1, H), full),
            pl.BlockSpec((1, H), full),
            pl.BlockSpec((H, H), full),
            pl.BlockSpec((1, H), full),
        ],
        out_specs=[
            pl.BlockSpec((R, H), row),
            pl.BlockSpec((R, H), row),
        ],
        out_shape=[
            jax.ShapeDtypeStruct((N, H), jnp.float32),
            jax.ShapeDtypeStruct((N, H), jnp.float32),
        ],
    )(s0, s1, x, wiht, gh_row, bih, lng, lnb, wnt, bn)


def kernel(x, edge_index, W_agg, b_agg, W_agg_r, b_agg_r, Wih, Whh, bih, bhh,
           Wih_r, Whh_r, bih_r, bhh_r, ln_g, ln_b):
    src = edge_index[0]
    dst = edge_index[1]
    zeros = jnp.zeros((STRIPE, H), jnp.float32)

    gshp = (NW, EPW)
    sshp = (NW, NCHUNK, CHUNK)
    ds_f, sg_f = lax.sort([dst, src], num_keys=1, is_stable=False)
    ds_r, sg_r = lax.sort([src, dst], num_keys=1, is_stable=False)
    sg_f, ds_f = sg_f.reshape(gshp), ds_f.reshape(sshp)
    sg_r, ds_r = sg_r.reshape(gshp), ds_r.reshape(sshp)

    wiht_f, wiht_r = Wih.T, Wih_r.T
    whht_f, whht_r = Whh.T, Whh_r.T
    wnt_f, wnt_r = W_agg.T, W_agg_r.T
    bih_f, bhh_f = bih.reshape(1, G3), bhh.reshape(1, G3)
    bih_r2, bhh_r2 = bih_r.reshape(1, G3), bhh_r.reshape(1, G3)
    lng, lnb = ln_g.reshape(1, H), ln_b.reshape(1, H)
    bn_f, bn_r = b_agg.reshape(1, H), b_agg_r.reshape(1, H)

    m = jnp.broadcast_to(W_agg.sum(axis=1) + b_agg, (N, H))
    gh_row = (Whh.sum(axis=1) + bhh).reshape(1, G3)

    p = _sc_scatter(m, sg_f, ds_f, zeros)
    h, m = _tc_step1(p[:N], p[N:], x, wiht_f, gh_row,
                     bih_f, lng, lnb, wnt_r, bn_r)
    p = _sc_scatter(m, sg_r, ds_r, zeros)
    h, m = _tc_step(p[:N], p[N:], x, h, wiht_r, whht_r,
                    bih_r2, bhh_r2, lng, lnb, wnt_f, bn_f)
    for _ in range(ROUNDS - 1):
        p = _sc_scatter(m, sg_f, ds_f, zeros)
        h, m = _tc_step(p[:N], p[N:], x, h, wiht_f, whht_f,
                        bih_f, bhh_f, lng, lnb, wnt_r, bn_r)
        p = _sc_scatter(m, sg_r, ds_r, zeros)
        h, m = _tc_step(p[:N], p[N:], x, h, wiht_r, whht_r,
                        bih_r2, bhh_r2, lng, lnb, wnt_f, bn_f)
    return h


# speedup vs baseline: 5.5965x; 1.0362x over previous
import functools

import jax
import jax.numpy as jnp
from jax import lax
from jax.experimental import pallas as pl
from jax.experimental.pallas import tpu as pltpu
from jax.experimental.pallas import tpu_sc as plsc

N = 10000
E = 320000
H = 128
F = 128
ROUNDS = 3

NC = 2
NS = 16
NW = NC * NS
EPW = E // NW
CHUNK = 80
NCHUNK = EPW // CHUNK
STRIPE = 624
TAIL = N - NS * STRIPE
ZROWS = 104


def _sc_scatter_make():
    mesh = plsc.VectorSubcoreMesh(core_axis_name="c", subcore_axis_name="s")

    @functools.partial(
        pl.kernel,
        out_type=jax.ShapeDtypeStruct((NC * N, H), jnp.float32),
        name="edge_scatter_add",
        mesh=mesh,
        scratch_types=[
            pltpu.VMEM((EPW,), jnp.int32),
            pltpu.VMEM((NCHUNK, CHUNK), jnp.int32),
            pltpu.VMEM((CHUNK, H), jnp.float32),
            pltpu.VMEM((CHUNK, H), jnp.float32),
            pltpu.VMEM_SHARED((N, H), jnp.float32),
            pltpu.SemaphoreType.DMA,
            pltpu.SemaphoreType.DMA,
        ],
    )
    def sc_scatter(m_hbm, gidx_hbm, sidx_hbm, zeros_hbm, out_hbm,
                   gi_v, si_v, rows0, rows1, acc, sem0, sem1):
        c = lax.axis_index("c")
        s = lax.axis_index("s")
        wid = s * NC + c

        pltpu.sync_copy(gidx_hbm.at[wid], gi_v)
        pltpu.sync_copy(sidx_hbm.at[wid], si_v)

        pltpu.sync_copy(zeros_hbm, acc.at[pl.ds(s * STRIPE, STRIPE)])

        @pl.when(s == NS - 1)
        def _zero_tail():
            pltpu.sync_copy(zeros_hbm.at[pl.ds(0, TAIL)],
                            acc.at[pl.ds(NS * STRIPE, TAIL)])

        plsc.subcore_barrier()

        bufs = (rows0, rows1)
        sems = (sem0, sem1)

        def start(g, buf, sem):
            pltpu.async_copy(m_hbm.at[gi_v.at[pl.ds(g * CHUNK, CHUNK)]],
                             buf, sem)

        def finish(g, buf, sem):
            pltpu.make_async_copy(m_hbm.at[gi_v.at[pl.ds(g * CHUNK, CHUNK)]],
                                  buf, sem).wait()
            pltpu.sync_copy(buf, acc.at[si_v.at[g]], add=True)

        start(0, bufs[0], sems[0])

        def body(i, carry):
            for b in range(2):
                g = 2 * i + b
                start(g + 1, bufs[1 - b], sems[1 - b])
                finish(g, bufs[b], sems[b])
            return carry

        lax.fori_loop(0, (NCHUNK - 1) // 2, body, 0)
        finish(NCHUNK - 1, bufs[0], sems[0])
        plsc.subcore_barrier()

        pltpu.sync_copy(acc.at[pl.ds(s * STRIPE, STRIPE)],
                        out_hbm.at[pl.ds(c * N + s * STRIPE, STRIPE)])

        @pl.when(s == NS - 1)
        def _copy_tail():
            pltpu.sync_copy(acc.at[pl.ds(NS * STRIPE, TAIL)],
                            out_hbm.at[pl.ds(c * N + NS * STRIPE, TAIL)])

    return sc_scatter


_sc_scatter = _sc_scatter_make()


R = 2000
G3 = 3 * H


def _rowsum(v):
    acc = v[:, 0:8] * 0.0
    for j in range(16):
        acc = acc + v[:, j * 8:(j + 1) * 8]
    a4 = acc[:, 0:4] + acc[:, 4:8]
    a2 = a4[:, 0:2] + a4[:, 2:4]
    return a2[:, 0:1] + a2[:, 1:2]


def _lnorm(hn, lng, lnb):
    mu = _rowsum(hn) * (1.0 / H)
    var = _rowsum((hn - mu) ** 2) * (1.0 / H)
    return (hn - mu) / jnp.sqrt(var + 1e-5) * lng + lnb


def _tc_step_body(s0_ref, s1_ref, x_ref, h_ref, wih_ref, whh_ref,
                  bih_ref, bhh_ref, lng_ref, lnb_ref, wn_ref, bn_ref,
                  h_out_ref, m_out_ref):
    msg = s0_ref[...] + s1_ref[...]
    x = x_ref[...]
    h = h_ref[...]
    xin = jnp.concatenate([msg, x], axis=-1)
    gi = jnp.dot(xin, wih_ref[...]) + bih_ref[...]
    gh = jnp.dot(h, whh_ref[...]) + bhh_ref[...]
    r = jax.nn.sigmoid(gi[:, :H] + gh[:, :H])
    z = jax.nn.sigmoid(gi[:, H:2 * H] + gh[:, H:2 * H])
    n = jnp.tanh(gi[:, 2 * H:] + r * gh[:, 2 * H:])
    hn = (1.0 - z) * n + z * h
    hn = _lnorm(hn, lng_ref[...], lnb_ref[...])
    h_out_ref[...] = hn
    m_out_ref[...] = jnp.dot(hn, wn_ref[...]) + bn_ref[...]


def _tc_step(s0, s1, x, h, wiht, whht, bih, bhh, lng, lnb, wnt, bn):
    row = lambda i: (i, 0)
    full = lambda i: (0, 0)
    return pl.pallas_call(
        _tc_step_body,
        grid=(N // R,),
        in_specs=[
            pl.BlockSpec((R, H), row),
            pl.BlockSpec((R, H), row),
            pl.BlockSpec((R, F), row),
            pl.BlockSpec((R, H), row),
            pl.BlockSpec((H + F, G3), full),
            pl.BlockSpec((H, G3), full),
            pl.BlockSpec((1, G3), full),
            pl.BlockSpec((1, G3), full),
            pl.BlockSpec((1, H), full),
            pl.BlockSpec((1, H), full),
            pl.BlockSpec((H, H), full),
            pl.BlockSpec((1, H), full),
        ],
        out_specs=[
            pl.BlockSpec((R, H), row),
            pl.BlockSpec((R, H), row),
        ],
        out_shape=[
            jax.ShapeDtypeStruct((N, H), jnp.float32),
            jax.ShapeDtypeStruct((N, H), jnp.float32),
        ],
    )(s0, s1, x, h, wiht, whht, bih, bhh, lng, lnb, wnt, bn)


def _tc_step1_body(s0_ref, s1_ref, x_ref, wih_ref, gh_ref, bih_ref,
                   lng_ref, lnb_ref, wn_ref, bn_ref, h_out_ref, m_out_ref):
    msg = s0_ref[...] + s1_ref[...]
    x = x_ref[...]
    xin = jnp.concatenate([msg, x], axis=-1)
    gi = jnp.dot(xin, wih_ref[...]) + bih_ref[...]
    gh = gh_ref[...]
    r = jax.nn.sigmoid(gi[:, :H] + gh[:, :H])
    z = jax.nn.sigmoid(gi[:, H:2 * H] + gh[:, H:2 * H])
    n = jnp.tanh(gi[:, 2 * H:] + r * gh[:, 2 * H:])
    hn = (1.0 - z) * n + z
    hn = _lnorm(hn, lng_ref[...], lnb_ref[...])
    h_out_ref[...] = hn
    m_out_ref[...] = jnp.dot(hn, wn_ref[...]) + bn_ref[...]


def _tc_step1(s0, s1, x, wiht, gh_row, bih, lng, lnb, wnt, bn):
    row = lambda i: (i, 0)
    full = lambda i: (0, 0)
    return pl.pallas_call(
        _tc_step1_body,
        grid=(N // R,),
        in_specs=[
            pl.BlockSpec((R, H), row),
            pl.BlockSpec((R, H), row),
            pl.BlockSpec((R, F), row),
            pl.BlockSpec((H + F, G3), full),
            pl.BlockSpec((1, G3), full),
            pl.BlockSpec((1, G3), full),
            pl.BlockSpec((1, H), full),
            pl.BlockSpec((1, H), full),
            pl.BlockSpec((H, H), full),
            pl.BlockSpec((1, H), full),
        ],
        out_specs=[
            pl.BlockSpec((R, H), row),
            pl.BlockSpec((R, H), row),
        ],
        out_shape=[
            jax.ShapeDtypeStruct((N, H), jnp.float32),
            jax.ShapeDtypeStruct((N, H), jnp.float32),
        ],
    )(s0, s1, x, wiht, gh_row, bih, lng, lnb, wnt, bn)


def kernel(x, edge_index, W_agg, b_agg, W_agg_r, b_agg_r, Wih, Whh, bih, bhh,
           Wih_r, Whh_r, bih_r, bhh_r, ln_g, ln_b):
    src = edge_index[0]
    dst = edge_index[1]
    zeros = jnp.zeros((STRIPE, H), jnp.float32)

    gshp = (NW, EPW)
    sshp = (NW, NCHUNK, CHUNK)
    ds_f, sg_f = lax.sort([dst, src], num_keys=1, is_stable=False)
    ds_r, sg_r = lax.sort([src, dst], num_keys=1, is_stable=False)
    sg_f, ds_f = sg_f.reshape(gshp), ds_f.reshape(sshp)
    sg_r, ds_r = sg_r.reshape(gshp), ds_r.reshape(sshp)

    wiht_f, wiht_r = Wih.T, Wih_r.T
    whht_f, whht_r = Whh.T, Whh_r.T
    wnt_f, wnt_r = W_agg.T, W_agg_r.T
    bih_f, bhh_f = bih.reshape(1, G3), bhh.reshape(1, G3)
    bih_r2, bhh_r2 = bih_r.reshape(1, G3), bhh_r.reshape(1, G3)
    lng, lnb = ln_g.reshape(1, H), ln_b.reshape(1, H)
    bn_f, bn_r = b_agg.reshape(1, H), b_agg_r.reshape(1, H)

    m = jnp.broadcast_to(W_agg.sum(axis=1) + b_agg, (N, H))
    gh_row = (Whh.sum(axis=1) + bhh).reshape(1, G3)

    p = _sc_scatter(m, sg_f, ds_f, zeros)
    h, m = _tc_step1(p[:N], p[N:], x, wiht_f, gh_row,
                     bih_f, lng, lnb, wnt_r, bn_r)
    p = _sc_scatter(m, sg_r, ds_r, zeros)
    h, m = _tc_step(p[:N], p[N:], x, h, wiht_r, whht_r,
                    bih_r2, bhh_r2, lng, lnb, wnt_f, bn_f)
    for _ in range(ROUNDS - 1):
        p = _sc_scatter(m, sg_f, ds_f, zeros)
        h, m = _tc_step(p[:N], p[N:], x, h, wiht_f, whht_f,
                        bih_f, bhh_f, lng, lnb, wnt_r, bn_r)
        p = _sc_scatter(m, sg_r, ds_r, zeros)
        h, m = _tc_step(p[:N], p[N:], x, h, wiht_r, whht_r,
                        bih_r2, bhh_r2, lng, lnb, wnt_f, bn_f)
    return h
